# baseline (speedup 1.0000x reference)
import functools

import jax
import jax.numpy as jnp
from jax import lax
from jax.experimental import pallas as pl
from jax.experimental.pallas import tpu as pltpu
from jax.experimental.pallas import tpu_sc as plsc

N = 10000
E = 160000
DIN = 256
DH = 512
DL = 128
HF = 128
NT = 16
EPT = E // NT
K = 80
NCH = EPT // K
NST = 5
GCH = NCH // NST
RSTRIDE = 624
RCOPY = 640

BM = 400
NBLK = N // BM

_sc_mesh = plsc.VectorSubcoreMesh(core_axis_name="c", subcore_axis_name="s")


def _seg_body(tab, sidx, dstr, zf, agg,
              src2, dst2, buf0, buf1, buf2, acc,
              gsem0, gsem1, gsem2, ssem0, ssem1, ssem2):
    cid = lax.axis_index("c")
    sid = lax.axis_index("s")
    rbase = sid * RSTRIDE

    def striped_copy(src_ref, dst_ref, dst_off=0):
        pltpu.sync_copy(src_ref.at[pl.ds(rbase, RCOPY)],
                        dst_ref.at[pl.ds(dst_off + rbase, RCOPY)])

    striped_copy(zf, acc)
    plsc.subcore_barrier()

    bufs = (buf0, buf1, buf2)
    gsems = (gsem0, gsem1, gsem2)
    ssems = (ssem0, ssem1, ssem2)

    def g_start(j, b):
        pltpu.async_copy(tab.at[src2.at[j]], bufs[b], gsems[b])

    def g_wait(b):
        pltpu.make_async_copy(tab.at[src2.at[0]], bufs[b], gsems[b]).wait()

    def s_start(j, b):
        pltpu.async_copy(bufs[b], acc.at[dst2.at[j]], ssems[b], add=True)

    def s_wait(b):
        pltpu.make_async_copy(bufs[b], acc.at[dst2.at[0]], ssems[b]).wait()

    def stage(st, carry):
        pltpu.sync_copy(sidx.at[(cid * NT + sid) * NST + st], src2)
        pltpu.sync_copy(dstr.at[sid * NST + st], dst2)
        g_start(0, 0)
        g_start(1, 1)
        g_wait(0); s_start(0, 0)
        g_start(2, 2)
        g_wait(1); s_start(1, 1)
        s_wait(0); g_start(3, 0)
        g_wait(2); s_start(2, 2)
        s_wait(1); g_start(4, 1)

        def triple(t, carry2):
            g_wait(0); s_start(3 * t, 0)
            s_wait(2); g_start(3 * t + 2, 2)
            g_wait(1); s_start(3 * t + 1, 1)
            s_wait(0); g_start(3 * t + 3, 0)
            g_wait(2); s_start(3 * t + 2, 2)
            s_wait(1); g_start(3 * t + 4, 1)
            return carry2

        lax.fori_loop(1, (GCH - 4) // 3, triple, 0)
        e = GCH - 4
        g_wait(0); s_start(e, 0)
        s_wait(2); g_start(e + 2, 2)
        g_wait(1); s_start(e + 1, 1)
        s_wait(0); g_start(e + 3, 0)
        g_wait(2); s_start(e + 2, 2)
        s_wait(1)
        g_wait(0); s_start(e + 3, 0)
        s_wait(2)
        s_wait(0)
        return carry

    lax.fori_loop(0, NST, stage, 0)
    plsc.subcore_barrier()

    striped_copy(acc, agg, dst_off=cid * N)


_seg = pl.kernel(
    _seg_body,
    mesh=_sc_mesh,
    out_type=[jax.ShapeDtypeStruct((2 * N, HF), jnp.float32)],
    scratch_types=[
        pltpu.VMEM((GCH, K), jnp.int32),
        pltpu.VMEM((GCH, K), jnp.int32),
        pltpu.VMEM((K, HF), jnp.float32),
        pltpu.VMEM((K, HF), jnp.float32),
        pltpu.VMEM((K, HF), jnp.float32),
        pltpu.VMEM_SHARED((N, HF), jnp.float32),
        pltpu.SemaphoreType.DMA,
        pltpu.SemaphoreType.DMA,
        pltpu.SemaphoreType.DMA,
        pltpu.SemaphoreType.DMA,
        pltpu.SemaphoreType.DMA,
        pltpu.SemaphoreType.DMA,
    ],
)

KD = 125
DCH = E // 2 // NT // KD


def _deg_body(dstr, ones_h, zf, deg, dst2, ones_v, dacc, gsem):
    cid = lax.axis_index("c")
    sid = lax.axis_index("s")
    rbase = sid * RSTRIDE

    def striped_copy(src_ref, dst_ref, dst_off=0):
        pltpu.sync_copy(src_ref.at[pl.ds(rbase, RCOPY)],
                        dst_ref.at[pl.ds(dst_off + rbase, RCOPY)])

    striped_copy(zf, dacc)
    pltpu.sync_copy(ones_h, ones_v)
    pltpu.sync_copy(dstr.at[cid * NT + sid], dst2)
    plsc.subcore_barrier()

    def s_start(j):
        pltpu.async_copy(ones_v, dacc.at[dst2.at[j]], gsem, add=True)

    def s_wait():
        pltpu.make_async_copy(ones_v, dacc.at[dst2.at[0]], gsem).wait()

    for j in range(4):
        s_start(j)

    def chunk(j, carry):
        s_start(j + 4)
        s_wait()
        return carry

    lax.fori_loop(0, DCH - 4, chunk, 0)
    for _ in range(4):
        s_wait()
    plsc.subcore_barrier()
    striped_copy(dacc, deg, dst_off=cid * N)


_deg_kernel = pl.kernel(
    _deg_body,
    mesh=_sc_mesh,
    out_type=[jax.ShapeDtypeStruct((2 * N, HF), jnp.float32)],
    scratch_types=[
        pltpu.VMEM((DCH, KD), jnp.int32),
        pltpu.VMEM((KD, HF), jnp.float32),
        pltpu.VMEM_SHARED((N, HF), jnp.float32),
        pltpu.SemaphoreType.DMA,
    ],
)


def _dot_t(a, w):
    return lax.dot_general(a, w, (((1,), (1,)), ((), ())),
                           preferred_element_type=jnp.float32)


def _tcr_body(x_r, wa_r, wb_r, o_r):
    o_r[...] = jnp.concatenate(
        [_dot_t(x_r[...], wa_r[...]), _dot_t(x_r[...], wb_r[...])], axis=1)


def _tcr(x, wa, wb):
    din = x.shape[1]
    do = wa.shape[0]
    return pl.pallas_call(
        _tcr_body,
        grid=(NBLK,),
        in_specs=[
            pl.BlockSpec((BM, din), lambda i: (i, 0)),
            pl.BlockSpec((do, din), lambda i: (0, 0)),
            pl.BlockSpec((do, din), lambda i: (0, 0)),
        ],
        out_specs=[pl.BlockSpec((BM, 2 * do), lambda i: (i, 0))],
        out_shape=[jax.ShapeDtypeStruct((N, 2 * do), jnp.float32)],
    )(x, wa, wb)


def _tc1_body(xr_r, al_r, ar_r, d0_r, d1_r, w1l_r, b1l_r, wmul_r,
              wlsl_r, h_r, p_r, deg_r):
    deg = d0_r[...][:, :1] + d1_r[...][:, :1]
    deg_r[...] = deg
    dinv = 1.0 / jnp.maximum(deg, 1.0)
    a = jnp.concatenate([al_r[...], ar_r[...]], axis=1) * dinv
    h = _dot_t(a, w1l_r[...]) + xr_r[...] + b1l_r[...]
    h = jnp.maximum(h, 0.0)
    h_r[...] = h
    p_r[...] = jnp.concatenate(
        [_dot_t(h, wmul_r[...]), _dot_t(h, wlsl_r[...])], axis=1)


def _tc1(xr, agg, deg2, w1l, b1l, wmul, wlsl):
    return pl.pallas_call(
        _tc1_body,
        grid=(NBLK,),
        in_specs=[
            pl.BlockSpec((BM, DH), lambda i: (i, 0)),
            pl.BlockSpec((BM, HF), lambda i: (i, 0)),
            pl.BlockSpec((BM, HF), lambda i: (i + NBLK, 0)),
            pl.BlockSpec((BM, HF), lambda i: (i, 0)),
            pl.BlockSpec((BM, HF), lambda i: (i + NBLK, 0)),
            pl.BlockSpec((DH, DIN), lambda i: (0, 0)),
            pl.BlockSpec((1, DH), lambda i: (0, 0)),
            pl.BlockSpec((DL, DH), lambda i: (0, 0)),
            pl.BlockSpec((DL, DH), lambda i: (0, 0)),
        ],
        out_specs=[
            pl.BlockSpec((BM, DH), lambda i: (i, 0)),
            pl.BlockSpec((BM, DIN), lambda i: (i, 0)),
            pl.BlockSpec((BM, 1), lambda i: (i, 0)),
        ],
        out_shape=[
            jax.ShapeDtypeStruct((N, DH), jnp.float32),
            jax.ShapeDtypeStruct((N, DIN), jnp.float32),
            jax.ShapeDtypeStruct((N, 1), jnp.float32),
        ],
    )(xr, agg, agg, deg2, deg2, w1l, b1l, wmul, wlsl)


def _tc2_body(r_r, am_r, as_r, deg_r, bmu_r, bls_r, eps_r, z_r, klp_r):
    dinv = 1.0 / jnp.maximum(deg_r[...], 1.0)
    mu = am_r[...] * dinv + bmu_r[...] + r_r[...][:, :DL]
    ls = as_r[...] * dinv + bls_r[...] + r_r[...][:, DL:]
    sigma = jnp.exp(ls)
    z_r[...] = mu + sigma * eps_r[...]
    part = jnp.sum(sigma * sigma + mu * mu - ls - 0.5)

    @pl.when(pl.program_id(0) == 0)
    def _():
        klp_r[0, 0] = 0.0

    klp_r[0, 0] += part


def _tc2(r, aggp, deg, bmu, bls, eps):
    return pl.pallas_call(
        _tc2_body,
        grid=(NBLK,),
        in_specs=[
            pl.BlockSpec((BM, DIN), lambda i: (i, 0)),
            pl.BlockSpec((BM, DL), lambda i: (i, 0)),
            pl.BlockSpec((BM, DL), lambda i: (i + NBLK, 0)),
            pl.BlockSpec((BM, 1), lambda i: (i, 0)),
            pl.BlockSpec((1, DL), lambda i: (0, 0)),
            pl.BlockSpec((1, DL), lambda i: (0, 0)),
            pl.BlockSpec((BM, DL), lambda i: (i, 0)),
        ],
        out_specs=[
            pl.BlockSpec((BM, DL), lambda i: (i, 0)),
            pl.BlockSpec(memory_space=pltpu.SMEM),
        ],
        out_shape=[
            jax.ShapeDtypeStruct((N, DL), jnp.float32),
            jax.ShapeDtypeStruct((1, 1), jnp.float32),
        ],
    )(r, aggp, aggp, deg, bmu, bls, eps)


def kernel(x, edge_index, W1_l, b1_l, W1_r, Wmu_l, bmu_l, Wmu_r,
           Wls_l, bls_l, Wls_r, eps):
    src = edge_index[0].astype(jnp.int32)
    dst = edge_index[1].astype(jnp.int32).reshape(NT * NST, GCH, K)
    sidx = jnp.concatenate([2 * src, 2 * src + 1]).reshape(2 * NT * NST, GCH, K)
    xv = x.reshape(2 * N, HF)
    zf = jnp.zeros((N, HF), jnp.float32)
    ones_h = jnp.ones((KD, HF), jnp.float32)
    dstd = edge_index[1].astype(jnp.int32).reshape(2 * NT, DCH, KD)

    aggx, = _seg(xv, sidx, dst, zf)
    deg2, = _deg_kernel(dstd, ones_h, zf)
    xr, = _tcr(x, W1_r[:DH // 2], W1_r[DH // 2:])

    h, p, deg = _tc1(xr, aggx, deg2, W1_l, b1_l.reshape(1, DH), Wmu_l, Wls_l)

    aggp, = _seg(p.reshape(2 * N, HF), sidx, dst, zf)
    r, = _tcr(h, Wmu_r, Wls_r)

    z, klp = _tc2(r, aggp, deg, bmu_l.reshape(1, DL), bls_l.reshape(1, DL),
                  eps)
    return z, klp[0, 0]


# speedup vs baseline: 1.0013x; 1.0013x over previous
import functools

import jax
import jax.numpy as jnp
from jax import lax
from jax.experimental import pallas as pl
from jax.experimental.pallas import tpu as pltpu
from jax.experimental.pallas import tpu_sc as plsc

N = 10000
E = 160000
DIN = 256
DH = 512
DL = 128
HF = 128
NT = 16
EPT = E // NT
K = 80
NCH = EPT // K
NST = 5
GCH = NCH // NST
RSTRIDE = 624
RCOPY = 640

BM = 400
NBLK = N // BM

_sc_mesh = plsc.VectorSubcoreMesh(core_axis_name="c", subcore_axis_name="s")


def _seg_body(tab, sidx, dstr, zf, agg,
              src2, dst2, buf0, buf1, buf2, acc,
              gsem0, gsem1, gsem2, ssem0, ssem1, ssem2):
    cid = lax.axis_index("c")
    sid = lax.axis_index("s")
    rbase = sid * RSTRIDE

    def striped_copy(src_ref, dst_ref, dst_off=0):
        pltpu.sync_copy(src_ref.at[pl.ds(rbase, RCOPY)],
                        dst_ref.at[pl.ds(dst_off + rbase, RCOPY)])

    striped_copy(zf, acc)
    plsc.subcore_barrier()

    bufs = (buf0, buf1, buf2)
    gsems = (gsem0, gsem1, gsem2)
    ssems = (ssem0, ssem1, ssem2)

    def g_start(j, b):
        pltpu.async_copy(tab.at[src2.at[j]], bufs[b], gsems[b])

    def g_wait(b):
        pltpu.make_async_copy(tab.at[src2.at[0]], bufs[b], gsems[b]).wait()

    def s_start(j, b):
        pltpu.async_copy(bufs[b], acc.at[dst2.at[j]], ssems[b], add=True)

    def s_wait(b):
        pltpu.make_async_copy(bufs[b], acc.at[dst2.at[0]], ssems[b]).wait()

    def stage(st, carry):
        pltpu.sync_copy(sidx.at[(cid * NT + sid) * NST + st], src2)
        pltpu.sync_copy(dstr.at[sid * NST + st], dst2)
        g_start(0, 0)
        g_start(1, 1)
        g_wait(0); s_start(0, 0)
        g_start(2, 2)
        g_wait(1); s_start(1, 1)
        s_wait(0); g_start(3, 0)
        g_wait(2); s_start(2, 2)
        s_wait(1); g_start(4, 1)

        def triple(t, carry2):
            g_wait(0); s_start(3 * t, 0)
            s_wait(2); g_start(3 * t + 2, 2)
            g_wait(1); s_start(3 * t + 1, 1)
            s_wait(0); g_start(3 * t + 3, 0)
            g_wait(2); s_start(3 * t + 2, 2)
            s_wait(1); g_start(3 * t + 4, 1)
            return carry2

        lax.fori_loop(1, (GCH - 4) // 3, triple, 0)
        e = GCH - 4
        g_wait(0); s_start(e, 0)
        s_wait(2); g_start(e + 2, 2)
        g_wait(1); s_start(e + 1, 1)
        s_wait(0); g_start(e + 3, 0)
        g_wait(2); s_start(e + 2, 2)
        s_wait(1)
        g_wait(0); s_start(e + 3, 0)
        s_wait(2)
        s_wait(0)
        return carry

    lax.fori_loop(0, NST, stage, 0)
    plsc.subcore_barrier()

    striped_copy(acc, agg, dst_off=cid * N)


_seg = pl.kernel(
    _seg_body,
    mesh=_sc_mesh,
    out_type=[jax.ShapeDtypeStruct((2 * N, HF), jnp.float32)],
    scratch_types=[
        pltpu.VMEM((GCH, K), jnp.int32),
        pltpu.VMEM((GCH, K), jnp.int32),
        pltpu.VMEM((K, HF), jnp.float32),
        pltpu.VMEM((K, HF), jnp.float32),
        pltpu.VMEM((K, HF), jnp.float32),
        pltpu.VMEM_SHARED((N, HF), jnp.float32),
        pltpu.SemaphoreType.DMA,
        pltpu.SemaphoreType.DMA,
        pltpu.SemaphoreType.DMA,
        pltpu.SemaphoreType.DMA,
        pltpu.SemaphoreType.DMA,
        pltpu.SemaphoreType.DMA,
    ],
)

KD = 125
DCH = E // 2 // NT // KD


def _deg_body(dstr, ones_h, zf, deg, dst2, ones_v, dacc, gsem):
    cid = lax.axis_index("c")
    sid = lax.axis_index("s")
    rbase = sid * RSTRIDE

    def striped_copy(src_ref, dst_ref, dst_off=0):
        pltpu.sync_copy(src_ref.at[pl.ds(rbase, RCOPY)],
                        dst_ref.at[pl.ds(dst_off + rbase, RCOPY)])

    striped_copy(zf, dacc)
    pltpu.sync_copy(ones_h, ones_v)
    pltpu.sync_copy(dstr.at[cid * NT + sid], dst2)
    plsc.subcore_barrier()

    def s_start(j):
        pltpu.async_copy(ones_v, dacc.at[dst2.at[j]], gsem, add=True)

    def s_wait():
        pltpu.make_async_copy(ones_v, dacc.at[dst2.at[0]], gsem).wait()

    for j in range(4):
        s_start(j)

    def chunk(j, carry):
        s_start(j + 4)
        s_wait()
        return carry

    lax.fori_loop(0, DCH - 4, chunk, 0)
    for _ in range(4):
        s_wait()
    plsc.subcore_barrier()
    striped_copy(dacc, deg, dst_off=cid * N)


_deg_kernel = pl.kernel(
    _deg_body,
    mesh=_sc_mesh,
    out_type=[jax.ShapeDtypeStruct((2 * N, HF), jnp.float32)],
    scratch_types=[
        pltpu.VMEM((DCH, KD), jnp.int32),
        pltpu.VMEM((KD, HF), jnp.float32),
        pltpu.VMEM_SHARED((N, HF), jnp.float32),
        pltpu.SemaphoreType.DMA,
    ],
)


def _dot_t(a, w):
    return lax.dot_general(a, w, (((1,), (1,)), ((), ())),
                           preferred_element_type=jnp.float32)


def _tc1_body(x_r, al_r, ar_r, d0_r, d1_r, w1l_r, b1l_r, w1r_r, wmul_r,
              wlsl_r, h_r, p_r, deg_r):
    deg = d0_r[...][:, :1] + d1_r[...][:, :1]
    deg_r[...] = deg
    dinv = 1.0 / jnp.maximum(deg, 1.0)
    a = jnp.concatenate([al_r[...], ar_r[...]], axis=1) * dinv
    h = _dot_t(a, w1l_r[...]) + _dot_t(x_r[...], w1r_r[...]) + b1l_r[...]
    h = jnp.maximum(h, 0.0)
    h_r[...] = h
    p_r[...] = jnp.concatenate(
        [_dot_t(h, wmul_r[...]), _dot_t(h, wlsl_r[...])], axis=1)


def _tc1(x, agg, deg2, w1l, b1l, w1r, wmul, wlsl):
    return pl.pallas_call(
        _tc1_body,
        grid=(NBLK,),
        in_specs=[
            pl.BlockSpec((BM, DIN), lambda i: (i, 0)),
            pl.BlockSpec((BM, HF), lambda i: (i, 0)),
            pl.BlockSpec((BM, HF), lambda i: (i + NBLK, 0)),
            pl.BlockSpec((BM, HF), lambda i: (i, 0)),
            pl.BlockSpec((BM, HF), lambda i: (i + NBLK, 0)),
            pl.BlockSpec((DH, DIN), lambda i: (0, 0)),
            pl.BlockSpec((1, DH), lambda i: (0, 0)),
            pl.BlockSpec((DH, DIN), lambda i: (0, 0)),
            pl.BlockSpec((DL, DH), lambda i: (0, 0)),
            pl.BlockSpec((DL, DH), lambda i: (0, 0)),
        ],
        out_specs=[
            pl.BlockSpec((BM, DH), lambda i: (i, 0)),
            pl.BlockSpec((BM, DIN), lambda i: (i, 0)),
            pl.BlockSpec((BM, 1), lambda i: (i, 0)),
        ],
        out_shape=[
            jax.ShapeDtypeStruct((N, DH), jnp.float32),
            jax.ShapeDtypeStruct((N, DIN), jnp.float32),
            jax.ShapeDtypeStruct((N, 1), jnp.float32),
        ],
    )(x, agg, agg, deg2, deg2, w1l, b1l, w1r, wmul, wlsl)


def _tc2_body(h_r, am_r, as_r, deg_r, wmur_r, wlsr_r, bmu_r, bls_r, eps_r,
              z_r, klp_r):
    dinv = 1.0 / jnp.maximum(deg_r[...], 1.0)
    mu = am_r[...] * dinv + bmu_r[...] + _dot_t(h_r[...], wmur_r[...])
    ls = as_r[...] * dinv + bls_r[...] + _dot_t(h_r[...], wlsr_r[...])
    sigma = jnp.exp(ls)
    z_r[...] = mu + sigma * eps_r[...]
    part = jnp.sum(sigma * sigma + mu * mu - ls - 0.5)

    @pl.when(pl.program_id(0) == 0)
    def _():
        klp_r[0, 0] = 0.0

    klp_r[0, 0] += part


def _tc2(h, aggp, deg, wmur, wlsr, bmu, bls, eps):
    return pl.pallas_call(
        _tc2_body,
        grid=(NBLK,),
        in_specs=[
            pl.BlockSpec((BM, DH), lambda i: (i, 0)),
            pl.BlockSpec((BM, DL), lambda i: (i, 0)),
            pl.BlockSpec((BM, DL), lambda i: (i + NBLK, 0)),
            pl.BlockSpec((BM, 1), lambda i: (i, 0)),
            pl.BlockSpec((DL, DH), lambda i: (0, 0)),
            pl.BlockSpec((DL, DH), lambda i: (0, 0)),
            pl.BlockSpec((1, DL), lambda i: (0, 0)),
            pl.BlockSpec((1, DL), lambda i: (0, 0)),
            pl.BlockSpec((BM, DL), lambda i: (i, 0)),
        ],
        out_specs=[
            pl.BlockSpec((BM, DL), lambda i: (i, 0)),
            pl.BlockSpec(memory_space=pltpu.SMEM),
        ],
        out_shape=[
            jax.ShapeDtypeStruct((N, DL), jnp.float32),
            jax.ShapeDtypeStruct((1, 1), jnp.float32),
        ],
    )(h, aggp, aggp, deg, wmur, wlsr, bmu, bls, eps)


def kernel(x, edge_index, W1_l, b1_l, W1_r, Wmu_l, bmu_l, Wmu_r,
           Wls_l, bls_l, Wls_r, eps):
    src = edge_index[0].astype(jnp.int32)
    dst = edge_index[1].astype(jnp.int32).reshape(NT * NST, GCH, K)
    sidx = jnp.concatenate([2 * src, 2 * src + 1]).reshape(2 * NT * NST, GCH, K)
    xv = x.reshape(2 * N, HF)
    zf = jnp.zeros((N, HF), jnp.float32)
    ones_h = jnp.ones((KD, HF), jnp.float32)
    dstd = edge_index[1].astype(jnp.int32).reshape(2 * NT, DCH, KD)

    aggx, = _seg(xv, sidx, dst, zf)
    deg2, = _deg_kernel(dstd, ones_h, zf)

    h, p, deg = _tc1(x, aggx, deg2, W1_l, b1_l.reshape(1, DH), W1_r,
                     Wmu_l, Wls_l)

    aggp, = _seg(p.reshape(2 * N, HF), sidx, dst, zf)

    z, klp = _tc2(h, aggp, deg, Wmu_r, Wls_r, bmu_l.reshape(1, DL),
                  bls_l.reshape(1, DL), eps)
    return z, klp[0, 0]


# speedup vs baseline: 1.0021x; 1.0008x over previous
import jax
import jax.numpy as jnp
from jax import lax
from jax.experimental import pallas as pl
from jax.experimental.pallas import tpu as pltpu
from jax.experimental.pallas import tpu_sc as plsc

N = 10000
E = 160000
DIN = 256
DH = 512
DL = 128
HF = 128
NT = 16
EPT = E // NT
K = 80
NCH = EPT // K
NST = 5
GCH = NCH // NST
RSTRIDE = 624
RCOPY = 640

BM = 400
NBLK = N // BM

_sc_mesh = plsc.VectorSubcoreMesh(core_axis_name="c", subcore_axis_name="s")


def _seg_body(tab, sidx, dstr, zf, agg,
              src2, dst2, buf0, buf1, buf2, acc,
              gsem0, gsem1, gsem2, ssem0, ssem1, ssem2):
    cid = lax.axis_index("c")
    sid = lax.axis_index("s")
    rbase = sid * RSTRIDE

    def striped_copy(src_ref, dst_ref, dst_off=0):
        pltpu.sync_copy(src_ref.at[pl.ds(rbase, RCOPY)],
                        dst_ref.at[pl.ds(dst_off + rbase, RCOPY)])

    striped_copy(zf, acc)
    plsc.subcore_barrier()

    bufs = (buf0, buf1, buf2)
    gsems = (gsem0, gsem1, gsem2)
    ssems = (ssem0, ssem1, ssem2)

    def g_start(j, b):
        pltpu.async_copy(tab.at[src2.at[j]], bufs[b], gsems[b])

    def g_wait(b):
        pltpu.make_async_copy(tab.at[src2.at[0]], bufs[b], gsems[b]).wait()

    def s_start(j, b):
        pltpu.async_copy(bufs[b], acc.at[dst2.at[j]], ssems[b], add=True)

    def s_wait(b):
        pltpu.make_async_copy(bufs[b], acc.at[dst2.at[0]], ssems[b]).wait()

    def stage(st, carry):
        pltpu.sync_copy(sidx.at[(cid * NT + sid) * NST + st], src2)
        pltpu.sync_copy(dstr.at[sid * NST + st], dst2)
        g_start(0, 0)
        g_start(1, 1)
        g_wait(0); s_start(0, 0)
        g_start(2, 2)
        g_wait(1); s_start(1, 1)
        s_wait(0); g_start(3, 0)
        g_wait(2); s_start(2, 2)
        s_wait(1); g_start(4, 1)

        def triple(t, carry2):
            g_wait(0); s_start(3 * t, 0)
            s_wait(2); g_start(3 * t + 2, 2)
            g_wait(1); s_start(3 * t + 1, 1)
            s_wait(0); g_start(3 * t + 3, 0)
            g_wait(2); s_start(3 * t + 2, 2)
            s_wait(1); g_start(3 * t + 4, 1)
            return carry2

        lax.fori_loop(1, (GCH - 4) // 3, triple, 0)
        e = GCH - 4
        g_wait(0); s_start(e, 0)
        s_wait(2); g_start(e + 2, 2)
        g_wait(1); s_start(e + 1, 1)
        s_wait(0); g_start(e + 3, 0)
        g_wait(2); s_start(e + 2, 2)
        s_wait(1)
        g_wait(0); s_start(e + 3, 0)
        s_wait(2)
        s_wait(0)
        return carry

    lax.fori_loop(0, NST, stage, 0)
    plsc.subcore_barrier()

    striped_copy(acc, agg, dst_off=cid * N)


_seg = pl.kernel(
    _seg_body,
    mesh=_sc_mesh,
    out_type=[jax.ShapeDtypeStruct((2 * N, HF), jnp.float32)],
    scratch_types=[
        pltpu.VMEM((GCH, K), jnp.int32),
        pltpu.VMEM((GCH, K), jnp.int32),
        pltpu.VMEM((K, HF), jnp.float32),
        pltpu.VMEM((K, HF), jnp.float32),
        pltpu.VMEM((K, HF), jnp.float32),
        pltpu.VMEM_SHARED((N, HF), jnp.float32),
        pltpu.SemaphoreType.DMA,
        pltpu.SemaphoreType.DMA,
        pltpu.SemaphoreType.DMA,
        pltpu.SemaphoreType.DMA,
        pltpu.SemaphoreType.DMA,
        pltpu.SemaphoreType.DMA,
    ],
)

KD = 125
DCH = E // 2 // NT // KD


def _deg_body(dstr, ones_h, zf, deg, dst2, ones_v, dacc, gsem):
    cid = lax.axis_index("c")
    sid = lax.axis_index("s")
    rbase = sid * RSTRIDE

    def striped_copy(src_ref, dst_ref, dst_off=0):
        pltpu.sync_copy(src_ref.at[pl.ds(rbase, RCOPY)],
                        dst_ref.at[pl.ds(dst_off + rbase, RCOPY)])

    striped_copy(zf, dacc)
    pltpu.sync_copy(ones_h, ones_v)
    pltpu.sync_copy(dstr.at[cid * NT + sid], dst2)
    plsc.subcore_barrier()

    def s_start(j):
        pltpu.async_copy(ones_v, dacc.at[dst2.at[j]], gsem, add=True)

    def s_wait():
        pltpu.make_async_copy(ones_v, dacc.at[dst2.at[0]], gsem).wait()

    for j in range(4):
        s_start(j)

    def chunk(j, carry):
        s_start(j + 4)
        s_wait()
        return carry

    lax.fori_loop(0, DCH - 4, chunk, 0)
    for _ in range(4):
        s_wait()
    plsc.subcore_barrier()
    striped_copy(dacc, deg, dst_off=cid * N)


_deg_kernel = pl.kernel(
    _deg_body,
    mesh=_sc_mesh,
    out_type=[jax.ShapeDtypeStruct((2 * N, HF), jnp.float32)],
    scratch_types=[
        pltpu.VMEM((DCH, KD), jnp.int32),
        pltpu.VMEM((KD, HF), jnp.float32),
        pltpu.VMEM_SHARED((N, HF), jnp.float32),
        pltpu.SemaphoreType.DMA,
    ],
)


def _dot_t(a, w):
    return lax.dot_general(a, w, (((1,), (1,)), ((), ())),
                           preferred_element_type=jnp.float32)


def _tc1_body(x_r, al_r, ar_r, d0_r, d1_r, w1l_r, b1l_r, w1r_r, wmul_r,
              wlsl_r, h_r, p_r, deg_r):
    deg = d0_r[...][:, :1] + d1_r[...][:, :1]
    deg_r[...] = deg
    dinv = 1.0 / jnp.maximum(deg, 1.0)
    a = jnp.concatenate([al_r[...], ar_r[...]], axis=1) * dinv
    h = _dot_t(a, w1l_r[...]) + _dot_t(x_r[...], w1r_r[...]) + b1l_r[...]
    h = jnp.maximum(h, 0.0)
    h_r[...] = h
    p_r[...] = jnp.concatenate(
        [_dot_t(h, wmul_r[...]), _dot_t(h, wlsl_r[...])], axis=1)


def _tc1(x, agg, deg2, w1l, b1l, w1r, wmul, wlsl):
    return pl.pallas_call(
        _tc1_body,
        grid=(NBLK,),
        in_specs=[
            pl.BlockSpec((BM, DIN), lambda i: (i, 0)),
            pl.BlockSpec((BM, HF), lambda i: (i, 0)),
            pl.BlockSpec((BM, HF), lambda i: (i + NBLK, 0)),
            pl.BlockSpec((BM, HF), lambda i: (i, 0)),
            pl.BlockSpec((BM, HF), lambda i: (i + NBLK, 0)),
            pl.BlockSpec((DH, DIN), lambda i: (0, 0)),
            pl.BlockSpec((1, DH), lambda i: (0, 0)),
            pl.BlockSpec((DH, DIN), lambda i: (0, 0)),
            pl.BlockSpec((DL, DH), lambda i: (0, 0)),
            pl.BlockSpec((DL, DH), lambda i: (0, 0)),
        ],
        out_specs=[
            pl.BlockSpec((BM, DH), lambda i: (i, 0)),
            pl.BlockSpec((BM, DIN), lambda i: (i, 0)),
            pl.BlockSpec((BM, 1), lambda i: (i, 0)),
        ],
        out_shape=[
            jax.ShapeDtypeStruct((N, DH), jnp.float32),
            jax.ShapeDtypeStruct((N, DIN), jnp.float32),
            jax.ShapeDtypeStruct((N, 1), jnp.float32),
        ],
    )(x, agg, agg, deg2, deg2, w1l, b1l, w1r, wmul, wlsl)


def _tc2_body(h_r, am_r, as_r, deg_r, wmur_r, wlsr_r, bmu_r, bls_r, eps_r,
              z_r, klp_r):
    dinv = 1.0 / jnp.maximum(deg_r[...], 1.0)
    mu = am_r[...] * dinv + bmu_r[...] + _dot_t(h_r[...], wmur_r[...])
    ls = as_r[...] * dinv + bls_r[...] + _dot_t(h_r[...], wlsr_r[...])
    sigma = jnp.exp(ls)
    z_r[...] = mu + sigma * eps_r[...]
    part = jnp.sum(sigma * sigma + mu * mu - ls - 0.5)

    @pl.when(pl.program_id(0) == 0)
    def _():
        klp_r[0, 0] = 0.0

    klp_r[0, 0] += part


def _tc2(h, aggp, deg, wmur, wlsr, bmu, bls, eps):
    return pl.pallas_call(
        _tc2_body,
        grid=(NBLK,),
        in_specs=[
            pl.BlockSpec((BM, DH), lambda i: (i, 0)),
            pl.BlockSpec((BM, DL), lambda i: (i, 0)),
            pl.BlockSpec((BM, DL), lambda i: (i + NBLK, 0)),
            pl.BlockSpec((BM, 1), lambda i: (i, 0)),
            pl.BlockSpec((DL, DH), lambda i: (0, 0)),
            pl.BlockSpec((DL, DH), lambda i: (0, 0)),
            pl.BlockSpec((1, DL), lambda i: (0, 0)),
            pl.BlockSpec((1, DL), lambda i: (0, 0)),
            pl.BlockSpec((BM, DL), lambda i: (i, 0)),
        ],
        out_specs=[
            pl.BlockSpec((BM, DL), lambda i: (i, 0)),
            pl.BlockSpec(memory_space=pltpu.SMEM),
        ],
        out_shape=[
            jax.ShapeDtypeStruct((N, DL), jnp.float32),
            jax.ShapeDtypeStruct((1, 1), jnp.float32),
        ],
    )(h, aggp, aggp, deg, wmur, wlsr, bmu, bls, eps)


def kernel(x, edge_index, W1_l, b1_l, W1_r, Wmu_l, bmu_l, Wmu_r,
           Wls_l, bls_l, Wls_r, eps):
    src = edge_index[0].astype(jnp.int32)
    dst = edge_index[1].astype(jnp.int32).reshape(NT * NST, GCH, K)
    sidx = jnp.concatenate([2 * src, 2 * src + 1]).reshape(2 * NT * NST, GCH, K)
    xv = x.reshape(2 * N, HF)
    zf = jnp.zeros((N, HF), jnp.float32)
    ones_h = jnp.ones((KD, HF), jnp.float32)
    dstd = edge_index[1].astype(jnp.int32).reshape(2 * NT, DCH, KD)

    aggx, = _seg(xv, sidx, dst, zf)
    deg2, = _deg_kernel(dstd, ones_h, zf)

    h, p, deg = _tc1(x, aggx, deg2, W1_l, b1_l.reshape(1, DH), W1_r,
                     Wmu_l, Wls_l)

    aggp, = _seg(p.reshape(2 * N, HF), sidx, dst, zf)

    z, klp = _tc2(h, aggp, deg, Wmu_r, Wls_r, bmu_l.reshape(1, DL),
                  bls_l.reshape(1, DL), eps)
    return z, klp[0, 0]
